# baseline (speedup 1.0000x reference)
import functools
import math

import jax
import jax.numpy as jnp
from jax import lax
from jax.experimental import pallas as pl
from jax.experimental.pallas import tpu as pltpu

_THRESHOLD = 0.8
_CAP_FACTOR = 1.25
_MIN_EXPERT_CAPACITY = 4
_LOSS_COEF = 0.01


def _routing_body(x_ref, wg_ref, coef_ref, aux_ref, acc_ref, *, T, TBLK, C, B, E):
    b = pl.program_id(0)
    i = pl.program_id(1)
    nblk = T // TBLK

    @pl.when(i == 0)
    def _init():
        acc_ref[0:3, :] = jnp.zeros((3, acc_ref.shape[1]), jnp.float32)

        @pl.when(b == 0)
        def _init_aux():
            acc_ref[3:4, :] = jnp.zeros((1, acc_ref.shape[1]), jnp.float32)

    x_blk = x_ref[0]
    wg = wg_ref[...]
    logits = jnp.dot(x_blk, wg, preferred_element_type=jnp.float32)
    m = jnp.max(logits, axis=1, keepdims=True)
    ex = jnp.exp(logits - m)
    p = ex / jnp.sum(ex, axis=1, keepdims=True)

    iota_e = lax.broadcasted_iota(jnp.int32, (TBLK, E), 1)
    rank = jnp.zeros((TBLK, E), jnp.float32)
    csum = jnp.zeros((TBLK, E), jnp.float32)
    for j in range(E):
        pj = p[:, j:j + 1]
        gt = pj > p
        eq = pj == p
        rank = rank + jnp.where(gt | (eq & (iota_e > j)), 1.0, 0.0)
        csum = csum + jnp.where(gt | (eq & (iota_e >= j)), pj, 0.0)
    keep = (csum < _THRESHOLD) | (rank == 0.0)
    maskf = keep.astype(jnp.float32)
    renorm = jnp.clip(jnp.sum(p * maskf, axis=1, keepdims=True), 1e-9, None)
    weight = p * maskf / renorm

    it0 = lax.broadcasted_iota(jnp.int32, (TBLK, TBLK), 0)
    it1 = lax.broadcasted_iota(jnp.int32, (TBLK, TBLK), 1)
    ltm = (it1 < it0).astype(jnp.float32)
    carry = acc_ref[0:1, 0:E]
    pos = lax.dot_general(
        ltm, maskf, (((1,), (0,)), ((), ())),
        preferred_element_type=jnp.float32,
    ) + carry
    acc_ref[0:1, 0:E] = carry + jnp.sum(maskf, axis=0, keepdims=True)
    keep_cap = (pos < float(C)) & keep
    coef_ref[0] = weight * keep_cap.astype(jnp.float32)

    acc_ref[1:2, 0:E] = acc_ref[1:2, 0:E] + jnp.sum(maskf, axis=0, keepdims=True)
    acc_ref[2:3, 0:E] = acc_ref[2:3, 0:E] + jnp.sum(p, axis=0, keepdims=True)

    @pl.when(i == nblk - 1)
    def _finish_b():
        partial = jnp.sum(acc_ref[1:2, 0:E] * acc_ref[2:3, 0:E]).reshape(1, 1)
        acc_ref[3:4, 0:1] = acc_ref[3:4, 0:1] + partial

        @pl.when(b == B - 1)
        def _emit():
            scale = (E * E * _LOSS_COEF) / (float(T) * float(T) * B * E)
            aux_ref[0:1, 0:1] = acc_ref[3:4, 0:1] * scale


def _ffn_body(x_ref, w1_ref, w2_ref, coef_ref, out_ref):
    e = pl.program_id(1)

    @pl.when(e == 0)
    def _init():
        out_ref[...] = jnp.zeros_like(out_ref)

    h = jnp.dot(x_ref[...], w1_ref[0], preferred_element_type=jnp.float32)
    h = jnp.maximum(h, 0.0).astype(jnp.bfloat16)
    y = jnp.dot(h, w2_ref[0], preferred_element_type=jnp.float32)
    cf = coef_ref[...]
    lane = lax.broadcasted_iota(jnp.int32, cf.shape, 1)
    col = jnp.sum(jnp.where(lane == e, cf, 0.0), axis=1, keepdims=True)
    out_ref[...] += col * y


def _routing(x, w_gating, *, interpret=False):
    B, T, D = x.shape
    E = w_gating.shape[-1]
    C = max(min(T, math.ceil(T * _CAP_FACTOR / E)), _MIN_EXPERT_CAPACITY)
    TBLK = 512
    nblk = T // TBLK
    coef, aux = pl.pallas_call(
        functools.partial(_routing_body, T=T, TBLK=TBLK, C=C, B=B, E=E),
        grid=(B, nblk),
        in_specs=[
            pl.BlockSpec((1, TBLK, D), lambda b, i: (b, i, 0)),
            pl.BlockSpec((D, E), lambda b, i: (0, 0)),
        ],
        out_specs=[
            pl.BlockSpec((1, TBLK, E), lambda b, i: (b, i, 0)),
            pl.BlockSpec((1, 1), lambda b, i: (0, 0)),
        ],
        out_shape=[
            jax.ShapeDtypeStruct((B, T, E), jnp.float32),
            jax.ShapeDtypeStruct((1, 1), jnp.float32),
        ],
        scratch_shapes=[pltpu.VMEM((8, 128), jnp.float32)],
        interpret=interpret,
    )(x, w_gating)
    return coef, aux, C


def _dense_ffn(x2, coef2, w1, w2, *, interpret=False):
    N, D = x2.shape
    E, _, H = w1.shape
    NBLK = 2048
    nb = N // NBLK
    xb = x2.astype(jnp.bfloat16)
    w1b = w1.astype(jnp.bfloat16)
    w2b = w2.astype(jnp.bfloat16)
    out = pl.pallas_call(
        _ffn_body,
        grid=(nb, E),
        in_specs=[
            pl.BlockSpec((NBLK, D), lambda i, e: (i, 0)),
            pl.BlockSpec((1, D, H), lambda i, e: (e, 0, 0)),
            pl.BlockSpec((1, H, D), lambda i, e: (e, 0, 0)),
            pl.BlockSpec((NBLK, E), lambda i, e: (i, 0)),
        ],
        out_specs=pl.BlockSpec((NBLK, D), lambda i, e: (i, 0)),
        out_shape=jax.ShapeDtypeStruct((N, D), jnp.float32),
        interpret=interpret,
    )(xb, w1b, w2b, coef2)
    return out


def kernel(inputs, w_gating, w1, w2):
    B, T, D = inputs.shape
    E = w_gating.shape[-1]
    coef, aux, _C = _routing(inputs, w_gating)
    coef2 = coef.reshape(B * T, E)
    x2 = inputs.reshape(B * T, D)
    out = _dense_ffn(x2, coef2, w1, w2)
    return out.reshape(B, T, D), aux.reshape(())
